# Initial kernel scaffold; baseline (speedup 1.0000x reference)
import jax
import jax.numpy as jnp
from jax.experimental import pallas as pl


def kernel(dino_features, W1, b1, W2, b2):
    raise NotImplementedError("write your pallas kernel here")


# speedup vs baseline: 1.1985x; 1.1985x over previous
import jax
import jax.numpy as jnp
from jax.experimental import pallas as pl
from jax.experimental.pallas import tpu as pltpu


def _saliency_body(x_ref, w1_ref, b1_ref, w2_ref, b2_ref, o_ref, xpad_ref):
    @pl.when(pl.program_id(0) == 0)
    def _():
        xpad_ref[...] = jnp.zeros_like(xpad_ref)

    xpad_ref[1:33, 1:33, :] = x_ref[0].astype(jnp.bfloat16)
    acc = jnp.zeros((1024, 128), jnp.float32)
    for k in range(9):
        dy, dx = k // 3, k % 3
        xs = xpad_ref[dy:dy + 32, dx:dx + 32, :].reshape(1024, 384)
        acc = acc + jnp.dot(xs, w1_ref[k], preferred_element_type=jnp.float32)
    h = jnp.maximum(acc + b1_ref[0][None, :], 0.0)
    logits = jnp.sum(h * w2_ref[0][None, :], axis=1, keepdims=True) + b2_ref[0, 0]
    o_ref[0] = jax.nn.sigmoid(logits).reshape(32, 32, 1)


def kernel(dino_features, W1, b1, W2, b2):
    B, H, W, C = dino_features.shape
    O = W1.shape[0]
    w1 = jnp.transpose(W1, (2, 3, 1, 0)).reshape(9, C, O).astype(jnp.bfloat16)
    w2 = W2.reshape(1, O)
    b1r = b1.reshape(1, O)
    b2r = b2.reshape(1, 1)

    out = pl.pallas_call(
        _saliency_body,
        grid=(B,),
        in_specs=[
            pl.BlockSpec((1, H, W, C), lambda b: (b, 0, 0, 0)),
            pl.BlockSpec((9, C, O), lambda b: (0, 0, 0)),
            pl.BlockSpec((1, O), lambda b: (0, 0)),
            pl.BlockSpec((1, O), lambda b: (0, 0)),
            pl.BlockSpec((1, 1), lambda b: (0, 0)),
        ],
        out_specs=pl.BlockSpec((1, H, W, 1), lambda b: (b, 0, 0, 0)),
        out_shape=jax.ShapeDtypeStruct((B, H, W, 1), jnp.float32),
        scratch_shapes=[pltpu.VMEM((H + 2, W + 2, C), jnp.bfloat16)],
    )(dino_features, w1, b1r, w2, b2r)
    return out


# speedup vs baseline: 1.3118x; 1.0945x over previous
import jax
import jax.numpy as jnp
from jax.experimental import pallas as pl
from jax.experimental.pallas import tpu as pltpu

_HALO = 48


def _saliency_body(x_ref, w1_ref, b1_ref, w2_ref, b2_ref, o_ref, s_ref, e_ref):
    @pl.when(pl.program_id(0) == 0)
    def _():
        s_ref[...] = jnp.zeros_like(s_ref)
        e_ref[...] = jnp.zeros_like(e_ref)

    x2d = x_ref[0].reshape(1024, 384)
    zrow = jnp.zeros((1, 384), jnp.float32)
    x_m = jnp.concatenate([zrow, x2d[:-1]], axis=0)
    x_p = jnp.concatenate([x2d[1:], zrow], axis=0)
    s_ref[0, _HALO:_HALO + 1024, :] = x_m.astype(jnp.bfloat16)
    s_ref[1, _HALO:_HALO + 1024, :] = x2d.astype(jnp.bfloat16)
    s_ref[2, _HALO:_HALO + 1024, :] = x_p.astype(jnp.bfloat16)

    e_ref[0, 16:47, :] = x_ref[0, :31, 31, :].astype(jnp.bfloat16)
    e_ref[1, 17:48, :] = x_ref[0, 1:, 0, :].astype(jnp.bfloat16)

    acc = jnp.zeros((1024, 128), jnp.float32)
    for k in range(9):
        dy, dx = k // 3, k % 3
        base = _HALO + (dy - 1) * 32
        xs = s_ref[dx, base:base + 1024, :]
        acc = acc + jnp.dot(xs, w1_ref[k], preferred_element_type=jnp.float32)

    corr_l = jnp.zeros((32, 128), jnp.float32)
    corr_r = jnp.zeros((32, 128), jnp.float32)
    for dy in range(3):
        corr_l = corr_l + jnp.dot(e_ref[0, 14 + dy:46 + dy, :], w1_ref[dy * 3],
                                  preferred_element_type=jnp.float32)
        corr_r = corr_r + jnp.dot(e_ref[1, 16 + dy:48 + dy, :], w1_ref[dy * 3 + 2],
                                  preferred_element_type=jnp.float32)

    acc3 = acc.reshape(32, 32, 128)
    col = jax.lax.broadcasted_iota(jnp.int32, (32, 32, 128), 1)
    fix = (jnp.where(col == 0, corr_l[:, None, :], 0.0)
           + jnp.where(col == 31, corr_r[:, None, :], 0.0))
    acc3 = acc3 - fix
    h = jnp.maximum(acc3 + b1_ref[0][None, None, :], 0.0)
    logits = jnp.sum(h * w2_ref[0][None, None, :], axis=2, keepdims=True) + b2_ref[0, 0]
    o_ref[0] = jax.nn.sigmoid(logits)


def kernel(dino_features, W1, b1, W2, b2):
    B, H, W, C = dino_features.shape
    O = W1.shape[0]
    w1 = jnp.transpose(W1, (2, 3, 1, 0)).reshape(9, C, O).astype(jnp.bfloat16)
    w2 = W2.reshape(1, O)
    b1r = b1.reshape(1, O)
    b2r = b2.reshape(1, 1)

    out = pl.pallas_call(
        _saliency_body,
        grid=(B,),
        in_specs=[
            pl.BlockSpec((1, H, W, C), lambda b: (b, 0, 0, 0)),
            pl.BlockSpec((9, C, O), lambda b: (0, 0, 0)),
            pl.BlockSpec((1, O), lambda b: (0, 0)),
            pl.BlockSpec((1, O), lambda b: (0, 0)),
            pl.BlockSpec((1, 1), lambda b: (0, 0)),
        ],
        out_specs=pl.BlockSpec((1, H, W, 1), lambda b: (b, 0, 0, 0)),
        out_shape=jax.ShapeDtypeStruct((B, H, W, 1), jnp.float32),
        scratch_shapes=[
            pltpu.VMEM((3, _HALO + 1024 + _HALO, C), jnp.bfloat16),
            pltpu.VMEM((2, 64, C), jnp.bfloat16),
        ],
    )(dino_features, w1, b1r, w2, b2r)
    return out


# speedup vs baseline: 1.3279x; 1.0122x over previous
import jax
import jax.numpy as jnp
from jax.experimental import pallas as pl
from jax.experimental.pallas import tpu as pltpu

_HALO = 48
_G = 4


def _saliency_body(x_ref, w1_ref, b1_ref, w2_ref, b2_ref, o_ref, s_ref, e_ref):
    @pl.when(pl.program_id(0) == 0)
    def _():
        s_ref[...] = jnp.zeros_like(s_ref)
        e_ref[...] = jnp.zeros_like(e_ref)

    for g in range(_G):
        x2d = x_ref[g].reshape(1024, 384)
        zrow = jnp.zeros((1, 384), jnp.float32)
        x_m = jnp.concatenate([zrow, x2d[:-1]], axis=0)
        x_p = jnp.concatenate([x2d[1:], zrow], axis=0)
        s_ref[0, _HALO:_HALO + 1024, :] = x_m.astype(jnp.bfloat16)
        s_ref[1, _HALO:_HALO + 1024, :] = x2d.astype(jnp.bfloat16)
        s_ref[2, _HALO:_HALO + 1024, :] = x_p.astype(jnp.bfloat16)

        e_ref[0, 16:47, :] = x_ref[g, :31, 31, :].astype(jnp.bfloat16)
        e_ref[1, 17:48, :] = x_ref[g, 1:, 0, :].astype(jnp.bfloat16)

        acc = jnp.zeros((1024, 128), jnp.float32)
        for k in range(9):
            dy, dx = k // 3, k % 3
            base = _HALO + (dy - 1) * 32
            xs = s_ref[dx, base:base + 1024, :]
            acc = acc + jnp.dot(xs, w1_ref[k], preferred_element_type=jnp.float32)

        corr_l = jnp.zeros((32, 128), jnp.float32)
        corr_r = jnp.zeros((32, 128), jnp.float32)
        for dy in range(3):
            corr_l = corr_l + jnp.dot(e_ref[0, 14 + dy:46 + dy, :],
                                      w1_ref[dy * 3],
                                      preferred_element_type=jnp.float32)
            corr_r = corr_r + jnp.dot(e_ref[1, 16 + dy:48 + dy, :],
                                      w1_ref[dy * 3 + 2],
                                      preferred_element_type=jnp.float32)

        acc3 = acc.reshape(32, 32, 128)
        col = jax.lax.broadcasted_iota(jnp.int32, (32, 32, 128), 1)
        fix = (jnp.where(col == 0, corr_l[:, None, :], 0.0)
               + jnp.where(col == 31, corr_r[:, None, :], 0.0))
        acc3 = acc3 - fix
        h = jnp.maximum(acc3 + b1_ref[0][None, None, :], 0.0)
        logits = (jnp.sum(h * w2_ref[0][None, None, :], axis=2, keepdims=True)
                  + b2_ref[0, 0])
        o_ref[g] = jax.nn.sigmoid(logits)


def kernel(dino_features, W1, b1, W2, b2):
    B, H, W, C = dino_features.shape
    O = W1.shape[0]
    w1 = jnp.transpose(W1, (2, 3, 1, 0)).reshape(9, C, O).astype(jnp.bfloat16)
    w2 = W2.reshape(1, O)
    b1r = b1.reshape(1, O)
    b2r = b2.reshape(1, 1)

    out = pl.pallas_call(
        _saliency_body,
        grid=(B // _G,),
        in_specs=[
            pl.BlockSpec((_G, H, W, C), lambda b: (b, 0, 0, 0)),
            pl.BlockSpec((9, C, O), lambda b: (0, 0, 0)),
            pl.BlockSpec((1, O), lambda b: (0, 0)),
            pl.BlockSpec((1, O), lambda b: (0, 0)),
            pl.BlockSpec((1, 1), lambda b: (0, 0)),
        ],
        out_specs=pl.BlockSpec((_G, H, W, 1), lambda b: (b, 0, 0, 0)),
        out_shape=jax.ShapeDtypeStruct((B, H, W, 1), jnp.float32),
        scratch_shapes=[
            pltpu.VMEM((3, _HALO + 1024 + _HALO, C), jnp.bfloat16),
            pltpu.VMEM((2, 64, C), jnp.bfloat16),
        ],
    )(dino_features, w1, b1r, w2, b2r)
    return out


# speedup vs baseline: 1.4239x; 1.0723x over previous
import jax
import jax.numpy as jnp
from jax.experimental import pallas as pl
from jax.experimental.pallas import tpu as pltpu

_HALO = 48
_G = 4


def _saliency_body(x_ref, w1_ref, b1_ref, w2_ref, b2_ref, o_ref, s_ref):
    @pl.when(pl.program_id(0) == 0)
    def _():
        s_ref[...] = jnp.zeros_like(s_ref)

    for g in range(_G):
        x2d = x_ref[g].reshape(1024, 384)
        zrow = jnp.zeros((1, 384), jnp.float32)
        row = jax.lax.broadcasted_iota(jnp.int32, (1024, 384), 0)
        x_m = jnp.where(row % 32 == 0, 0.0,
                        jnp.concatenate([zrow, x2d[:-1]], axis=0))
        x_p = jnp.where(row % 32 == 31, 0.0,
                        jnp.concatenate([x2d[1:], zrow], axis=0))
        s_ref[_HALO:_HALO + 1024, 0:384] = x_m.astype(jnp.bfloat16)
        s_ref[_HALO:_HALO + 1024, 384:768] = x2d.astype(jnp.bfloat16)
        s_ref[_HALO:_HALO + 1024, 768:1152] = x_p.astype(jnp.bfloat16)

        acc = jnp.zeros((1024, 128), jnp.float32)
        for dy in range(3):
            base = _HALO + (dy - 1) * 32
            acc = acc + jnp.dot(s_ref[base:base + 1024, :], w1_ref[dy],
                                preferred_element_type=jnp.float32)

        h = jnp.maximum(acc + b1_ref[0][None, :], 0.0)
        logits = (jnp.sum(h * w2_ref[0][None, :], axis=1, keepdims=True)
                  + b2_ref[0, 0])
        o_ref[g] = jax.nn.sigmoid(logits).reshape(32, 32, 1)


def kernel(dino_features, W1, b1, W2, b2):
    B, H, W, C = dino_features.shape
    O = W1.shape[0]
    w1 = jnp.transpose(W1, (2, 3, 1, 0)).reshape(3, 3 * C, O).astype(jnp.bfloat16)
    w2 = W2.reshape(1, O)
    b1r = b1.reshape(1, O)
    b2r = b2.reshape(1, 1)

    out = pl.pallas_call(
        _saliency_body,
        grid=(B // _G,),
        in_specs=[
            pl.BlockSpec((_G, H, W, C), lambda b: (b, 0, 0, 0)),
            pl.BlockSpec((3, 3 * C, O), lambda b: (0, 0, 0)),
            pl.BlockSpec((1, O), lambda b: (0, 0)),
            pl.BlockSpec((1, O), lambda b: (0, 0)),
            pl.BlockSpec((1, 1), lambda b: (0, 0)),
        ],
        out_specs=pl.BlockSpec((_G, H, W, 1), lambda b: (b, 0, 0, 0)),
        out_shape=jax.ShapeDtypeStruct((B, H, W, 1), jnp.float32),
        scratch_shapes=[
            pltpu.VMEM((_HALO + 1024 + _HALO, 3 * C), jnp.bfloat16),
        ],
    )(dino_features, w1, b1r, w2, b2r)
    return out
